# Initial kernel scaffold; baseline (speedup 1.0000x reference)
import jax
import jax.numpy as jnp
from jax.experimental import pallas as pl


def kernel(x, gate_W, gate_b, fc1_W, fc1_b, fc2_W, fc2_b):
    raise NotImplementedError("write your pallas kernel here")


# speedup vs baseline: 6.9770x; 6.9770x over previous
import functools
import math

import jax
import jax.numpy as jnp
from jax import lax
from jax.experimental import pallas as pl
from jax.experimental.pallas import tpu as pltpu
from jax.experimental.pallas import tpu_sc as plsc

B, S, D, E, K, H, O = 1, 2048, 768, 64, 1, 768, 768
T = 128
NT = S // T
G = NT + E - 1


def _router_body(x_ref, gw_ref, gb_ref, eid_ref, aux_ref):
    logits = lax.dot_general(
        x_ref[...], gw_ref[...], (((1,), (1,)), ((), ())),
        preferred_element_type=jnp.float32) + gb_ref[...]
    eid_ref[...] = jnp.argmax(logits, axis=1).astype(jnp.int32)
    mx = jnp.max(logits, axis=1, keepdims=True)
    lse = jnp.log(jnp.sum(jnp.exp(logits - mx), axis=1, keepdims=True)) + mx
    mean_logprob = jnp.sum(logits) / (S * E) - jnp.sum(lse) / S
    aux_ref[0, 0] = 0.01 * (1.0 / E) * (
        jnp.log(jnp.float32(1.0 / E)) - mean_logprob)


def _route(x2, gate_W, gate_b):
    return pl.pallas_call(
        _router_body,
        out_shape=[
            jax.ShapeDtypeStruct((S,), jnp.int32),
            jax.ShapeDtypeStruct((1, 1), jnp.float32),
        ],
        out_specs=[
            pl.BlockSpec(memory_space=pltpu.VMEM),
            pl.BlockSpec(memory_space=pltpu.SMEM),
        ],
    )(x2, gate_W, gate_b.reshape(1, E))


def _schedule(eid):
    i32 = jnp.int32
    sort_idx = jnp.argsort(eid, stable=True).astype(i32)
    sorted_e = eid[sort_idx]
    erange = jnp.arange(E, dtype=i32)
    starts = jnp.searchsorted(sorted_e, erange, side='left').astype(i32)
    ends = jnp.searchsorted(sorted_e, erange, side='right').astype(i32)
    counts = ends - starts
    nonempty = counts > 0
    first_t = starts // T
    last_t = jnp.where(nonempty, (ends - 1) // T, 0)
    w = jnp.where(nonempty, last_t - first_t + 1, 0)
    cum = jnp.cumsum(w)
    total = cum[-1]
    cum0 = cum - w
    last_e = jnp.max(jnp.where(nonempty, erange, -1)).astype(i32)

    g_ids = jnp.arange(G, dtype=i32)
    active = g_ids < total
    eo_g = jnp.minimum(jnp.searchsorted(cum, g_ids, side='right'), E - 1)
    eo = jnp.where(active, eo_g, last_e).astype(i32)
    to = jnp.where(active, first_t[eo_g] + (g_ids - cum0[eo_g]), NT - 1)
    to = to.astype(i32)
    base = to * T
    lo = jnp.where(active, jnp.maximum(starts[eo], base) - base, 0).astype(i32)
    hi = jnp.where(active, jnp.minimum(ends[eo], base + T) - base, 0).astype(i32)

    inv = jnp.zeros((S,), i32).at[sort_idx].set(jnp.arange(S, dtype=i32))
    return sort_idx, inv, eo, to, lo, hi


def _ffn_body(eo_ref, to_ref, lo_ref, hi_ref,
              xs_ref, w1_ref, b1_ref, w2_ref, b2_ref, out_ref):
    g = pl.program_id(0)
    h = lax.dot_general(xs_ref[...], w1_ref[0], (((1,), (1,)), ((), ())),
                        preferred_element_type=jnp.float32) + b1_ref[0]
    h = 0.5 * h * (1.0 + lax.erf(h * (1.0 / math.sqrt(2.0))))
    y = lax.dot_general(h, w2_ref[0], (((1,), (1,)), ((), ())),
                        preferred_element_type=jnp.float32) + b2_ref[0]
    rows = lax.broadcasted_iota(jnp.int32, (T, 1), 0)
    m = (rows >= lo_ref[g]) & (rows < hi_ref[g])
    contrib = jnp.where(m, y, 0.0)
    prev = to_ref[jnp.maximum(g - 1, 0)]
    first = jnp.logical_or(g == 0, to_ref[g] != prev)

    @pl.when(first)
    def _():
        out_ref[...] = contrib

    @pl.when(jnp.logical_not(first))
    def _():
        out_ref[...] = out_ref[...] + contrib


def _ffn(eo, to, lo, hi, x_sorted, fc1_W, fc1_b, fc2_W, fc2_b):
    grid_spec = pltpu.PrefetchScalarGridSpec(
        num_scalar_prefetch=4,
        grid=(G,),
        in_specs=[
            pl.BlockSpec((T, D), lambda g, eo, to, lo, hi: (to[g], 0)),
            pl.BlockSpec((1, H, D), lambda g, eo, to, lo, hi: (eo[g], 0, 0)),
            pl.BlockSpec((1, 1, H), lambda g, eo, to, lo, hi: (eo[g], 0, 0)),
            pl.BlockSpec((1, O, H), lambda g, eo, to, lo, hi: (eo[g], 0, 0)),
            pl.BlockSpec((1, 1, O), lambda g, eo, to, lo, hi: (eo[g], 0, 0)),
        ],
        out_specs=pl.BlockSpec((T, O), lambda g, eo, to, lo, hi: (to[g], 0)),
    )
    return pl.pallas_call(
        _ffn_body,
        grid_spec=grid_spec,
        out_shape=jax.ShapeDtypeStruct((S, O), jnp.float32),
    )(eo, to, lo, hi, x_sorted, fc1_W, fc1_b.reshape(E, 1, H),
      fc2_W, fc2_b.reshape(E, 1, O))


def _sc_gather(table, idx):
    info = plsc.get_sparse_core_info()
    nw = info.num_cores * info.num_subcores
    n, d = table.shape
    b_per_w = n // nw
    mesh = plsc.VectorSubcoreMesh(core_axis_name="c", subcore_axis_name="s")

    @functools.partial(
        pl.kernel, mesh=mesh,
        out_type=jax.ShapeDtypeStruct((n, d), jnp.float32),
        scratch_types=[
            pltpu.VMEM((b_per_w,), jnp.int32),
            pltpu.VMEM((b_per_w, d), jnp.float32),
            pltpu.SemaphoreType.DMA,
        ],
    )
    def gk(table_hbm, idx_hbm, out_hbm, idx_v, rows_v, sem):
        wid = lax.axis_index("s") * info.num_cores + lax.axis_index("c")
        base = wid * b_per_w
        pltpu.sync_copy(idx_hbm.at[pl.ds(base, b_per_w)], idx_v)
        pltpu.async_copy(table_hbm.at[idx_v], rows_v, sem).wait()
        pltpu.sync_copy(rows_v, out_hbm.at[pl.ds(base, b_per_w)])

    return gk(table, idx)


def kernel(x, gate_W, gate_b, fc1_W, fc1_b, fc2_W, fc2_b):
    x2 = x.reshape(S, D)
    eid, aux = _route(x2, gate_W, gate_b)
    sort_idx, inv, eo, to, lo, hi = _schedule(eid)
    x_sorted = _sc_gather(x2, sort_idx)
    out_sorted = _ffn(eo, to, lo, hi, x_sorted, fc1_W, fc1_b, fc2_W, fc2_b)
    out = _sc_gather(out_sorted, inv)
    return out.reshape(B, S, O), aux.reshape(())


# speedup vs baseline: 9.2803x; 1.3301x over previous
import functools
import math

import jax
import jax.numpy as jnp
from jax import lax
from jax.experimental import pallas as pl
from jax.experimental.pallas import tpu as pltpu
from jax.experimental.pallas import tpu_sc as plsc

B, S, D, E, K, H, O = 1, 2048, 768, 64, 1, 768, 768
T = 128
NT = S // T
G = NT + E - 1
GP = 128
CH = 128
NCH = S // CH


def _router_body(x_ref, gw_ref, gb_ref, pos_ref, eo_ref, to_ref, lo_ref,
                 hi_ref, aux_ref):
    f32, i32 = jnp.float32, jnp.int32
    logits = lax.dot_general(
        x_ref[...], gw_ref[...], (((1,), (1,)), ((), ())),
        preferred_element_type=f32) + gb_ref[...]

    mx = jnp.max(logits, axis=1, keepdims=True)
    lse = jnp.log(jnp.sum(jnp.exp(logits - mx), axis=1, keepdims=True)) + mx
    mean_logprob = jnp.sum(logits) / (S * E) - jnp.sum(lse) / S
    aux_ref[0, 0] = 0.01 * (1.0 / E) * (
        jnp.log(jnp.float32(1.0 / E)) - mean_logprob)

    u_strict = (lax.broadcasted_iota(i32, (E, E), 0)
                < lax.broadcasted_iota(i32, (E, E), 1)).astype(f32)
    u_incl = (lax.broadcasted_iota(i32, (E, E), 0)
              <= lax.broadcasted_iota(i32, (E, E), 1)).astype(f32)
    oh_raw = (logits == mx).astype(f32)
    ex1 = lax.dot_general(oh_raw, u_strict, (((1,), (0,)), ((), ())),
                          preferred_element_type=f32)
    oh = oh_raw * (ex1 == 0.0).astype(f32)

    l_strict = (lax.broadcasted_iota(i32, (CH, CH), 0)
                > lax.broadcasted_iota(i32, (CH, CH), 1)).astype(f32)
    run = jnp.zeros((1, E), f32)
    ranks = []
    for c in range(NCH):
        ohc = oh[c * CH:(c + 1) * CH]
        r = lax.dot_general(l_strict, ohc, (((1,), (0,)), ((), ())),
                            preferred_element_type=f32)
        ranks.append(r + run)
        run = run + jnp.sum(ohc, axis=0, keepdims=True)
    rank2d = jnp.concatenate(ranks, axis=0)
    counts = run

    offs = lax.dot_general(counts, u_strict, (((1,), (0,)), ((), ())),
                           preferred_element_type=f32)
    rank_tok = jnp.sum(rank2d * oh, axis=1)
    offs_tok = jnp.sum(oh * offs, axis=1)
    pos_ref[...] = (offs_tok + rank_tok).astype(i32)

    starts = offs
    ends = offs + counts
    nonempty = counts > 0.0
    first_t = jnp.floor(starts * (1.0 / T))
    last_t = jnp.where(nonempty, jnp.floor((ends - 1.0) * (1.0 / T)), 0.0)
    w = jnp.where(nonempty, last_t - first_t + 1.0, 0.0)
    cum_w = lax.dot_general(w, u_incl, (((1,), (0,)), ((), ())),
                            preferred_element_type=f32)
    cum0 = cum_w - w
    total = cum_w[0, E - 1]
    erow = lax.broadcasted_iota(i32, (1, E), 1).astype(f32)
    last_e = jnp.max(jnp.where(nonempty, erow, -1.0))

    gcol = lax.broadcasted_iota(i32, (GP, 1), 0).astype(f32)
    active = gcol < total
    eo_g = jnp.minimum(
        jnp.sum((cum_w <= gcol).astype(f32), axis=1, keepdims=True),
        float(E - 1))
    ohg = (eo_g == lax.broadcasted_iota(i32, (GP, E), 1).astype(f32)
           ).astype(f32)
    f_g = jnp.sum(ohg * first_t, axis=1, keepdims=True)
    cum0_g = jnp.sum(ohg * cum0, axis=1, keepdims=True)
    starts_g = jnp.sum(ohg * starts, axis=1, keepdims=True)
    ends_g = jnp.sum(ohg * ends, axis=1, keepdims=True)
    to = jnp.where(active, f_g + (gcol - cum0_g), float(NT - 1))
    tbase = to * float(T)
    lo = jnp.where(active, jnp.maximum(starts_g, tbase) - tbase, 0.0)
    hi = jnp.where(active, jnp.minimum(ends_g, tbase + float(T)) - tbase, 0.0)
    eo = jnp.where(active, eo_g, last_e)

    eo_ref[...] = eo[:G, 0].astype(i32)
    to_ref[...] = to[:G, 0].astype(i32)
    lo_ref[...] = lo[:G, 0].astype(i32)
    hi_ref[...] = hi[:G, 0].astype(i32)


def _route(x2, gate_W, gate_b):
    i32 = jnp.int32
    return pl.pallas_call(
        _router_body,
        out_shape=[
            jax.ShapeDtypeStruct((S,), i32),
            jax.ShapeDtypeStruct((G,), i32),
            jax.ShapeDtypeStruct((G,), i32),
            jax.ShapeDtypeStruct((G,), i32),
            jax.ShapeDtypeStruct((G,), i32),
            jax.ShapeDtypeStruct((1, 1), jnp.float32),
        ],
        out_specs=[
            pl.BlockSpec(memory_space=pltpu.VMEM),
            pl.BlockSpec(memory_space=pltpu.VMEM),
            pl.BlockSpec(memory_space=pltpu.VMEM),
            pl.BlockSpec(memory_space=pltpu.VMEM),
            pl.BlockSpec(memory_space=pltpu.VMEM),
            pl.BlockSpec(memory_space=pltpu.SMEM),
        ],
    )(x2, gate_W, gate_b.reshape(1, E))


def _ffn_body(eo_ref, to_ref, lo_ref, hi_ref,
              xs_ref, w1_ref, b1_ref, w2_ref, b2_ref, out_ref):
    g = pl.program_id(0)
    h = lax.dot_general(xs_ref[...], w1_ref[0], (((1,), (1,)), ((), ())),
                        preferred_element_type=jnp.float32) + b1_ref[0]
    h = 0.5 * h * (1.0 + lax.erf(h * (1.0 / math.sqrt(2.0))))
    y = lax.dot_general(h, w2_ref[0], (((1,), (1,)), ((), ())),
                        preferred_element_type=jnp.float32) + b2_ref[0]
    rows = lax.broadcasted_iota(jnp.int32, (T, 1), 0)
    m = (rows >= lo_ref[g]) & (rows < hi_ref[g])
    contrib = jnp.where(m, y, 0.0)
    prev = to_ref[jnp.maximum(g - 1, 0)]
    first = jnp.logical_or(g == 0, to_ref[g] != prev)

    @pl.when(first)
    def _():
        out_ref[...] = contrib

    @pl.when(jnp.logical_not(first))
    def _():
        out_ref[...] = out_ref[...] + contrib


def _ffn(eo, to, lo, hi, x_sorted, fc1_W, fc1_b, fc2_W, fc2_b):
    grid_spec = pltpu.PrefetchScalarGridSpec(
        num_scalar_prefetch=4,
        grid=(G,),
        in_specs=[
            pl.BlockSpec((T, D), lambda g, eo, to, lo, hi: (to[g], 0)),
            pl.BlockSpec((1, H, D), lambda g, eo, to, lo, hi: (eo[g], 0, 0)),
            pl.BlockSpec((1, 1, H), lambda g, eo, to, lo, hi: (eo[g], 0, 0)),
            pl.BlockSpec((1, O, H), lambda g, eo, to, lo, hi: (eo[g], 0, 0)),
            pl.BlockSpec((1, 1, O), lambda g, eo, to, lo, hi: (eo[g], 0, 0)),
        ],
        out_specs=pl.BlockSpec((T, O), lambda g, eo, to, lo, hi: (to[g], 0)),
    )
    return pl.pallas_call(
        _ffn_body,
        grid_spec=grid_spec,
        out_shape=jax.ShapeDtypeStruct((S, O), jnp.float32),
    )(eo, to, lo, hi, x_sorted, fc1_W, fc1_b.reshape(E, 1, H),
      fc2_W, fc2_b.reshape(E, 1, O))


def _sc_scatter(rows, pos):
    info = plsc.get_sparse_core_info()
    nw = info.num_cores * info.num_subcores
    n, d = rows.shape
    b_per_w = n // nw
    mesh = plsc.VectorSubcoreMesh(core_axis_name="c", subcore_axis_name="s")

    @functools.partial(
        pl.kernel, mesh=mesh,
        out_type=jax.ShapeDtypeStruct((n, d), jnp.float32),
        scratch_types=[
            pltpu.VMEM((b_per_w,), jnp.int32),
            pltpu.VMEM((b_per_w, d), jnp.float32),
            pltpu.SemaphoreType.DMA,
        ],
    )
    def sk(rows_hbm, pos_hbm, out_hbm, idx_v, rows_v, sem):
        wid = lax.axis_index("s") * info.num_cores + lax.axis_index("c")
        base = wid * b_per_w
        pltpu.sync_copy(pos_hbm.at[pl.ds(base, b_per_w)], idx_v)
        pltpu.sync_copy(rows_hbm.at[pl.ds(base, b_per_w)], rows_v)
        pltpu.async_copy(rows_v, out_hbm.at[idx_v], sem).wait()

    return sk(rows, pos)


def _sc_gather(table, idx):
    info = plsc.get_sparse_core_info()
    nw = info.num_cores * info.num_subcores
    n, d = table.shape
    b_per_w = n // nw
    mesh = plsc.VectorSubcoreMesh(core_axis_name="c", subcore_axis_name="s")

    @functools.partial(
        pl.kernel, mesh=mesh,
        out_type=jax.ShapeDtypeStruct((n, d), jnp.float32),
        scratch_types=[
            pltpu.VMEM((b_per_w,), jnp.int32),
            pltpu.VMEM((b_per_w, d), jnp.float32),
            pltpu.SemaphoreType.DMA,
        ],
    )
    def gk(table_hbm, idx_hbm, out_hbm, idx_v, rows_v, sem):
        wid = lax.axis_index("s") * info.num_cores + lax.axis_index("c")
        base = wid * b_per_w
        pltpu.sync_copy(idx_hbm.at[pl.ds(base, b_per_w)], idx_v)
        pltpu.async_copy(table_hbm.at[idx_v], rows_v, sem).wait()
        pltpu.sync_copy(rows_v, out_hbm.at[pl.ds(base, b_per_w)])

    return gk(table, idx)


def kernel(x, gate_W, gate_b, fc1_W, fc1_b, fc2_W, fc2_b):
    x2 = x.reshape(S, D)
    pos, eo, to, lo, hi, aux = _route(x2, gate_W, gate_b)
    x_sorted = _sc_scatter(x2, pos)
    out_sorted = _ffn(eo, to, lo, hi, x_sorted, fc1_W, fc1_b, fc2_W, fc2_b)
    out = _sc_gather(out_sorted, pos)
    return out.reshape(B, S, O), aux.reshape(())
